# baseline (speedup 1.0000x reference)
import functools

import jax
import jax.numpy as jnp
from jax import lax
from jax.experimental import pallas as pl
from jax.experimental.pallas import tpu as pltpu
from jax.experimental.pallas import tpu_sc as plsc

B, T, D, L = 16, 2048, 256, 4096
HALF = L // 2
G = 64
NCHUNK = HALF // G
NBUF = 6
LA = 3
MAXDUR = 8
LANES = 16

_mesh = plsc.VectorSubcoreMesh(core_axis_name="c", subcore_axis_name="s")


@functools.partial(
    pl.kernel,
    out_type=[
        jax.ShapeDtypeStruct((B, L, D), jnp.float32),
        jax.ShapeDtypeStruct((B, L), jnp.int32),
    ],
    mesh=_mesh,
    scratch_types=[
        pltpu.VMEM((T,), jnp.int32),
        pltpu.VMEM((HALF,), jnp.int32),
        *([pltpu.VMEM((G, D), jnp.float32)] * NBUF),
        pltpu.VMEM((HALF,), jnp.int32),
        pltpu.VMEM((LANES,), jnp.int32),
        *([pltpu.SemaphoreType.DMA] * (2 * NBUF)),
    ],
    compiler_params=pltpu.CompilerParams(needs_layout_passes=False),
)
def _length_regulator(x_hbm, dur_hbm, ml_hbm, out_hbm, mask_hbm,
                      dur_v, idx_v, *rest):
    rows = rest[:NBUF]
    mask_v = rest[NBUF]
    ml_v = rest[NBUF + 1]
    gsems = rest[NBUF + 2:NBUF + 2 + NBUF]
    wsems = rest[NBUF + 2 + NBUF:]
    c = lax.axis_index("c")
    s = lax.axis_index("s")
    wid = s * 2 + c
    b = wid // 2
    p0 = (wid % 2) * HALF

    pltpu.sync_copy(dur_hbm.at[b], dur_v)
    pltpu.sync_copy(ml_hbm, ml_v)

    zeros16 = jnp.zeros((LANES,), jnp.int32)
    iota = lax.iota(jnp.int32, LANES)

    def init_body(i, carry):
        idx_v[pl.ds(i * LANES, LANES)] = zeros16
        return carry

    lax.fori_loop(0, HALF // LANES, init_body, jnp.int32(0))

    def chunk_body(i, carry):
        d = dur_v[pl.ds(i * LANES, LANES)]
        incl = plsc.cumsum(d) + carry
        start = incl - d
        tok = b * T + i * LANES + iota
        rel = start - p0
        for step in range(MAXDUR - 1):
            pos = rel + step
            m = (d > step) & (pos >= 0) & (pos < HALF)
            plsc.store_scatter(idx_v, [pos], tok, mask=m)
        return carry + jnp.sum(d)

    total = lax.fori_loop(0, T // LANES, chunk_body, jnp.int32(0))
    ml_s = jnp.max(ml_v[...])
    eff = jnp.minimum(total, ml_s)

    onesf = jnp.ones((LANES,), jnp.float32)
    zerosf = jnp.zeros((LANES,), jnp.float32)

    def fire_gather(g, buf, gsem):
        return pltpu.async_copy(x_hbm.at[idx_v.at[pl.ds(g * G, G)]], buf, gsem)

    gd = [None] * NCHUNK
    wd = [None] * NCHUNK
    for g in range(LA):
        gd[g] = fire_gather(g, rows[g], gsems[g])

    def mask_body(i, carry):
        pos = p0 + i * LANES + iota
        mask_v[pl.ds(i * LANES, LANES)] = (pos >= eff).astype(jnp.int32)
        return carry

    lax.fori_loop(0, HALF // LANES, mask_body, jnp.int32(0))
    pltpu.sync_copy(mask_v, mask_hbm.at[b, pl.ds(p0, HALF)])

    for g in range(NCHUNK):
        p = g % NBUF
        if g + LA < NCHUNK:
            prev = g + LA - NBUF
            if prev >= 0:
                wd[prev].wait()
            q = (g + LA) % NBUF
            gd[g + LA] = fire_gather(g + LA, rows[q], gsems[q])
        gd[g].wait()
        c0 = p0 + g * G
        buf = rows[p]

        @pl.when(eff < c0 + G)
        def _mask_chunk():
            def row_body(r, carry):
                rowpos = lax.broadcast(c0 + r, (LANES,))
                scale = jnp.where(rowpos < eff, onesf, zerosf)
                for k in range(D // LANES):
                    sl = pl.ds(k * LANES, LANES)
                    buf[r, sl] = buf[r, sl] * scale
                return carry

            lax.fori_loop(0, G, row_body, jnp.int32(0))

        wd[g] = pltpu.async_copy(buf, out_hbm.at[b, pl.ds(c0, G)], wsems[p])
    for t in range(max(0, NCHUNK + LA - NBUF), NCHUNK):
        wd[t].wait()


def kernel(x, durations, max_length):
    xf = x.reshape(B * T, D)
    dur = durations.astype(jnp.int32)
    ml = jnp.full((LANES,), max_length, dtype=jnp.int32)
    out, mask_i32 = _length_regulator(xf, dur, ml)
    return out, mask_i32 != 0


# speedup vs baseline: 1.0286x; 1.0286x over previous
import functools

import jax
import jax.numpy as jnp
from jax import lax
from jax.experimental import pallas as pl
from jax.experimental.pallas import tpu as pltpu
from jax.experimental.pallas import tpu_sc as plsc

B, T, D, L = 16, 2048, 256, 4096
HALF = L // 2
G = 128
NCHUNK = HALF // G
NBUF = 3
LA = 2
MAXDUR = 8
LANES = 16

_mesh = plsc.VectorSubcoreMesh(core_axis_name="c", subcore_axis_name="s")


@functools.partial(
    pl.kernel,
    out_type=[
        jax.ShapeDtypeStruct((B, L, D), jnp.float32),
        jax.ShapeDtypeStruct((B, L), jnp.int32),
    ],
    mesh=_mesh,
    scratch_types=[
        pltpu.VMEM((T,), jnp.int32),
        pltpu.VMEM((HALF,), jnp.int32),
        *([pltpu.VMEM((G, D), jnp.float32)] * NBUF),
        pltpu.VMEM((HALF,), jnp.int32),
        pltpu.VMEM((LANES,), jnp.int32),
        *([pltpu.SemaphoreType.DMA] * (2 * NBUF)),
    ],
    compiler_params=pltpu.CompilerParams(needs_layout_passes=False),
)
def _length_regulator(x_hbm, dur_hbm, ml_hbm, out_hbm, mask_hbm,
                      dur_v, idx_v, *rest):
    rows = rest[:NBUF]
    mask_v = rest[NBUF]
    ml_v = rest[NBUF + 1]
    gsems = rest[NBUF + 2:NBUF + 2 + NBUF]
    wsems = rest[NBUF + 2 + NBUF:]
    c = lax.axis_index("c")
    s = lax.axis_index("s")
    wid = s * 2 + c
    b = wid // 2
    p0 = (wid % 2) * HALF

    pltpu.sync_copy(dur_hbm.at[b], dur_v)
    pltpu.sync_copy(ml_hbm, ml_v)

    zeros16 = jnp.zeros((LANES,), jnp.int32)
    iota = lax.iota(jnp.int32, LANES)

    def init_body(i, carry):
        idx_v[pl.ds(i * LANES, LANES)] = zeros16
        return carry

    lax.fori_loop(0, HALF // LANES, init_body, jnp.int32(0))

    def chunk_body(i, carry):
        d = dur_v[pl.ds(i * LANES, LANES)]
        incl = plsc.cumsum(d) + carry
        start = incl - d
        tok = b * T + i * LANES + iota
        rel = start - p0
        for step in range(MAXDUR - 1):
            pos = rel + step
            m = (d > step) & (pos >= 0) & (pos < HALF)
            plsc.store_scatter(idx_v, [pos], tok, mask=m)
        return carry + jnp.sum(d)

    total = lax.fori_loop(0, T // LANES, chunk_body, jnp.int32(0))
    ml_s = jnp.max(ml_v[...])
    eff = jnp.minimum(total, ml_s)

    onesf = jnp.ones((LANES,), jnp.float32)
    zerosf = jnp.zeros((LANES,), jnp.float32)

    def fire_gather(g, buf, gsem):
        return pltpu.async_copy(x_hbm.at[idx_v.at[pl.ds(g * G, G)]], buf, gsem)

    gd = [None] * NCHUNK
    wd = [None] * NCHUNK
    for g in range(LA):
        gd[g] = fire_gather(g, rows[g], gsems[g])

    def mask_body(i, carry):
        pos = p0 + i * LANES + iota
        mask_v[pl.ds(i * LANES, LANES)] = (pos >= eff).astype(jnp.int32)
        return carry

    lax.fori_loop(0, HALF // LANES, mask_body, jnp.int32(0))
    pltpu.sync_copy(mask_v, mask_hbm.at[b, pl.ds(p0, HALF)])

    for g in range(NCHUNK):
        p = g % NBUF
        if g + LA < NCHUNK:
            prev = g + LA - NBUF
            if prev >= 0:
                wd[prev].wait()
            q = (g + LA) % NBUF
            gd[g + LA] = fire_gather(g + LA, rows[q], gsems[q])
        gd[g].wait()
        c0 = p0 + g * G
        buf = rows[p]

        @pl.when(eff < c0 + G)
        def _mask_chunk():
            def row_body(r, carry):
                rowpos = lax.broadcast(c0 + r, (LANES,))
                scale = jnp.where(rowpos < eff, onesf, zerosf)
                for k in range(D // LANES):
                    sl = pl.ds(k * LANES, LANES)
                    buf[r, sl] = buf[r, sl] * scale
                return carry

            lax.fori_loop(0, G, row_body, jnp.int32(0))

        wd[g] = pltpu.async_copy(buf, out_hbm.at[b, pl.ds(c0, G)], wsems[p])
    for t in range(max(0, NCHUNK + LA - NBUF), NCHUNK):
        wd[t].wait()


def kernel(x, durations, max_length):
    xf = x.reshape(B * T, D)
    dur = durations.astype(jnp.int32)
    ml = jnp.full((LANES,), max_length, dtype=jnp.int32)
    out, mask_i32 = _length_regulator(xf, dur, ml)
    return out, mask_i32 != 0
